# baseline (speedup 1.0000x reference)
import functools

import jax
import jax.numpy as jnp
from jax import lax
from jax.experimental import pallas as pl
from jax.experimental.pallas import tpu as pltpu
from jax.experimental.pallas import tpu_sc as plsc

N_PAIR = 32
N_CTX = 8
MAX_L = 40
D = 768
NUM_BASE = 92
NUM_NOVEL = 40
N_CLS = NUM_BASE + NUM_NOVEL
VOCAB = 1000
D_ENTI = 256
SUF_L = MAX_L - 1 - N_CTX

C_BLK = 66
NCC = N_CLS // C_BLK

N_IDS = 2 * N_PAIR
G_WORKERS = 8
G_ROWS = N_IDS // G_WORKERS
TM_CHUNK = N_CLS * MAX_L

_SC_MESH = plsc.VectorSubcoreMesh(core_axis_name="c", subcore_axis_name="s")


@functools.partial(
    pl.kernel,
    mesh=_SC_MESH,
    out_type=(
        jax.ShapeDtypeStruct((N_IDS, D_ENTI), jnp.float32),
        jax.ShapeDtypeStruct((N_PAIR * TM_CHUNK,), jnp.int32),
    ),
    scratch_types=[
        pltpu.VMEM((G_ROWS,), jnp.int32),
        pltpu.VMEM((G_ROWS, D_ENTI), jnp.float32),
        pltpu.VMEM((TM_CHUNK,), jnp.int32),
        pltpu.SemaphoreType.DMA,
    ],
)
def _sc_gather(ids_hbm, enti_hbm, tm_hbm, rows_out, tm_out,
               idx_v, rows_v, tm_v, sem):
    wid = lax.axis_index("s") * 2 + lax.axis_index("c")
    pltpu.sync_copy(tm_hbm, tm_v)
    pltpu.sync_copy(tm_v, tm_out.at[pl.ds(wid * TM_CHUNK, TM_CHUNK)])

    @pl.when(wid < G_WORKERS)
    def _():
        base = wid * G_ROWS
        pltpu.sync_copy(ids_hbm.at[pl.ds(base, G_ROWS)], idx_v)
        pltpu.async_copy(enti_hbm.at[idx_v], rows_v, sem).wait()
        pltpu.sync_copy(rows_v, rows_out.at[pl.ds(base, G_ROWS)])


def _ctx_body(g_ref, sW1_ref, sb1_ref, sW2_ref, oW1_ref, ob1_ref,
              oW2_ref, sctx_in_ref, octx_in_ref, sctx_ref, octx_ref):
    s_e = g_ref[0:N_PAIR]
    o_e = g_ref[N_PAIR:N_IDS]
    s_h = jnp.maximum(
        jnp.dot(s_e, sW1_ref[:], preferred_element_type=jnp.float32) + sb1_ref[:], 0.0)
    o_h = jnp.maximum(
        jnp.dot(o_e, oW1_ref[:], preferred_element_type=jnp.float32) + ob1_ref[:], 0.0)
    s_emb = jnp.dot(s_h, sW2_ref[:], preferred_element_type=jnp.float32)
    o_emb = jnp.dot(o_h, oW2_ref[:], preferred_element_type=jnp.float32)
    sctx_ref[:] = sctx_in_ref[:][None, :, :] + s_emb[:, None, :]
    octx_ref[:] = octx_in_ref[:][None, :, :] + o_emb[:, None, :]


def _assemble_body(pre_ref, suf_ref, sctx_ref, octx_ref, subj_ref, obj_ref):
    pre = pre_ref[:]
    suf = suf_ref[:]
    s_ctx = jnp.broadcast_to(sctx_ref[0][None, :, :], (C_BLK, N_CTX, D))
    o_ctx = jnp.broadcast_to(octx_ref[0][None, :, :], (C_BLK, N_CTX, D))
    subj_ref[:, 0:1, :] = pre
    subj_ref[:, 1:1 + N_CTX, :] = s_ctx
    subj_ref[:, 1 + N_CTX:MAX_L, :] = suf
    obj_ref[:, 0:1, :] = pre
    obj_ref[:, 1:1 + N_CTX, :] = o_ctx
    obj_ref[:, 1 + N_CTX:MAX_L, :] = suf


def kernel(so_cls_ids, enti_txt_embds, prefix_embds, suffix_embds, token_mask,
           subj_ctx_embds, obj_ctx_embds, sW1, sb1, sW2, oW1, ob1, oW2):
    prefix_sl = prefix_embds[1:N_CLS + 1]
    suffix_sl = suffix_embds[1:N_CLS + 1]
    tm_flat = token_mask[1:N_CLS + 1].reshape(-1)
    ids_flat = so_cls_ids.T.reshape(-1)

    gathered, tm_rep_flat = _sc_gather(ids_flat, enti_txt_embds, tm_flat)

    s_ctx, o_ctx = pl.pallas_call(
        _ctx_body,
        out_shape=(
            jax.ShapeDtypeStruct((N_PAIR, N_CTX, D), jnp.float32),
            jax.ShapeDtypeStruct((N_PAIR, N_CTX, D), jnp.float32),
        ),
    )(gathered, sW1, sb1, sW2, oW1, ob1, oW2, subj_ctx_embds, obj_ctx_embds)

    subj, obj = pl.pallas_call(
        _assemble_body,
        grid=(NCC, N_PAIR),
        in_specs=[
            pl.BlockSpec((C_BLK, 1, D), lambda cc, p: (cc, 0, 0)),
            pl.BlockSpec((C_BLK, SUF_L, D), lambda cc, p: (cc, 0, 0)),
            pl.BlockSpec((1, N_CTX, D), lambda cc, p: (p, 0, 0)),
            pl.BlockSpec((1, N_CTX, D), lambda cc, p: (p, 0, 0)),
        ],
        out_specs=[
            pl.BlockSpec((C_BLK, MAX_L, D), lambda cc, p: (p * NCC + cc, 0, 0)),
            pl.BlockSpec((C_BLK, MAX_L, D), lambda cc, p: (p * NCC + cc, 0, 0)),
        ],
        out_shape=(
            jax.ShapeDtypeStruct((N_PAIR * N_CLS, MAX_L, D), jnp.float32),
            jax.ShapeDtypeStruct((N_PAIR * N_CLS, MAX_L, D), jnp.float32),
        ),
    )(prefix_sl, suffix_sl, s_ctx, o_ctx)

    return subj, obj, tm_rep_flat.reshape(N_PAIR * N_CLS, MAX_L)


# speedup vs baseline: 1.0001x; 1.0001x over previous
import functools

import jax
import jax.numpy as jnp
from jax import lax
from jax.experimental import pallas as pl
from jax.experimental.pallas import tpu as pltpu
from jax.experimental.pallas import tpu_sc as plsc

N_PAIR = 32
N_CTX = 8
MAX_L = 40
D = 768
NUM_BASE = 92
NUM_NOVEL = 40
N_CLS = NUM_BASE + NUM_NOVEL
VOCAB = 1000
D_ENTI = 256
SUF_L = MAX_L - 1 - N_CTX

C_BLK = 66
NCC = N_CLS // C_BLK

N_IDS = 2 * N_PAIR
G_WORKERS = 8
G_ROWS = N_IDS // G_WORKERS
TM_CHUNK = N_CLS * MAX_L

_SC_MESH = plsc.VectorSubcoreMesh(core_axis_name="c", subcore_axis_name="s")


@functools.partial(
    pl.kernel,
    mesh=_SC_MESH,
    out_type=jax.ShapeDtypeStruct((N_IDS, D_ENTI), jnp.float32),
    scratch_types=[
        pltpu.VMEM((G_ROWS,), jnp.int32),
        pltpu.VMEM((G_ROWS, D_ENTI), jnp.float32),
        pltpu.SemaphoreType.DMA,
    ],
)
def _sc_gather(ids_hbm, enti_hbm, rows_out, idx_v, rows_v, sem):
    wid = lax.axis_index("s") * 2 + lax.axis_index("c")
    @pl.when(wid < G_WORKERS)
    def _():
        base = wid * G_ROWS
        pltpu.sync_copy(ids_hbm.at[pl.ds(base, G_ROWS)], idx_v)
        pltpu.async_copy(enti_hbm.at[idx_v], rows_v, sem).wait()
        pltpu.sync_copy(rows_v, rows_out.at[pl.ds(base, G_ROWS)])


@functools.partial(
    pl.kernel,
    mesh=_SC_MESH,
    out_type=jax.ShapeDtypeStruct((N_PAIR * TM_CHUNK,), jnp.int32),
    scratch_types=[pltpu.VMEM((TM_CHUNK,), jnp.int32)],
)
def _sc_mask(tm_hbm, tm_out, tm_v):
    wid = lax.axis_index("s") * 2 + lax.axis_index("c")
    pltpu.sync_copy(tm_hbm, tm_v)
    pltpu.sync_copy(tm_v, tm_out.at[pl.ds(wid * TM_CHUNK, TM_CHUNK)])


def _ctx_body(g_ref, sW1_ref, sb1_ref, sW2_ref, oW1_ref, ob1_ref,
              oW2_ref, sctx_in_ref, octx_in_ref, sctx_ref, octx_ref):
    s_e = g_ref[0:N_PAIR]
    o_e = g_ref[N_PAIR:N_IDS]
    s_h = jnp.maximum(
        jnp.dot(s_e, sW1_ref[:], preferred_element_type=jnp.float32) + sb1_ref[:], 0.0)
    o_h = jnp.maximum(
        jnp.dot(o_e, oW1_ref[:], preferred_element_type=jnp.float32) + ob1_ref[:], 0.0)
    s_emb = jnp.dot(s_h, sW2_ref[:], preferred_element_type=jnp.float32)
    o_emb = jnp.dot(o_h, oW2_ref[:], preferred_element_type=jnp.float32)
    sctx_ref[:] = sctx_in_ref[:][None, :, :] + s_emb[:, None, :]
    octx_ref[:] = octx_in_ref[:][None, :, :] + o_emb[:, None, :]


def _assemble_body(pre_ref, suf_ref, sctx_ref, octx_ref, subj_ref, obj_ref):
    pre = pre_ref[:]
    suf = suf_ref[:]
    s_ctx = jnp.broadcast_to(sctx_ref[0][None, :, :], (C_BLK, N_CTX, D))
    o_ctx = jnp.broadcast_to(octx_ref[0][None, :, :], (C_BLK, N_CTX, D))
    subj_ref[:, 0:1, :] = pre
    subj_ref[:, 1:1 + N_CTX, :] = s_ctx
    subj_ref[:, 1 + N_CTX:MAX_L, :] = suf
    obj_ref[:, 0:1, :] = pre
    obj_ref[:, 1:1 + N_CTX, :] = o_ctx
    obj_ref[:, 1 + N_CTX:MAX_L, :] = suf


def kernel(so_cls_ids, enti_txt_embds, prefix_embds, suffix_embds, token_mask,
           subj_ctx_embds, obj_ctx_embds, sW1, sb1, sW2, oW1, ob1, oW2):
    prefix_sl = prefix_embds[1:N_CLS + 1]
    suffix_sl = suffix_embds[1:N_CLS + 1]
    tm_flat = token_mask[1:N_CLS + 1].reshape(-1)
    ids_flat = so_cls_ids.T.reshape(-1)

    gathered = _sc_gather(ids_flat, enti_txt_embds)
    tm_rep_flat = _sc_mask(tm_flat)

    s_ctx, o_ctx = pl.pallas_call(
        _ctx_body,
        out_shape=(
            jax.ShapeDtypeStruct((N_PAIR, N_CTX, D), jnp.float32),
            jax.ShapeDtypeStruct((N_PAIR, N_CTX, D), jnp.float32),
        ),
    )(gathered, sW1, sb1, sW2, oW1, ob1, oW2, subj_ctx_embds, obj_ctx_embds)

    subj, obj = pl.pallas_call(
        _assemble_body,
        grid=(NCC, N_PAIR),
        in_specs=[
            pl.BlockSpec((C_BLK, 1, D), lambda cc, p: (cc, 0, 0)),
            pl.BlockSpec((C_BLK, SUF_L, D), lambda cc, p: (cc, 0, 0)),
            pl.BlockSpec((1, N_CTX, D), lambda cc, p: (p, 0, 0)),
            pl.BlockSpec((1, N_CTX, D), lambda cc, p: (p, 0, 0)),
        ],
        out_specs=[
            pl.BlockSpec((C_BLK, MAX_L, D), lambda cc, p: (p * NCC + cc, 0, 0)),
            pl.BlockSpec((C_BLK, MAX_L, D), lambda cc, p: (p * NCC + cc, 0, 0)),
        ],
        out_shape=(
            jax.ShapeDtypeStruct((N_PAIR * N_CLS, MAX_L, D), jnp.float32),
            jax.ShapeDtypeStruct((N_PAIR * N_CLS, MAX_L, D), jnp.float32),
        ),
    )(prefix_sl, suffix_sl, s_ctx, o_ctx)

    return subj, obj, tm_rep_flat.reshape(N_PAIR * N_CLS, MAX_L)


# speedup vs baseline: 1.0052x; 1.0052x over previous
import functools

import jax
import jax.numpy as jnp
from jax import lax
from jax.experimental import pallas as pl
from jax.experimental.pallas import tpu as pltpu
from jax.experimental.pallas import tpu_sc as plsc

N_PAIR = 32
N_CTX = 8
MAX_L = 40
D = 768
NUM_BASE = 92
NUM_NOVEL = 40
N_CLS = NUM_BASE + NUM_NOVEL
VOCAB = 1000
D_ENTI = 256
SUF_L = MAX_L - 1 - N_CTX

C_BLK = 66
NCC = N_CLS // C_BLK

N_IDS = 2 * N_PAIR
G_WORKERS = 8
G_ROWS = N_IDS // G_WORKERS
TM_CHUNK = N_CLS * MAX_L

_SC_MESH = plsc.VectorSubcoreMesh(core_axis_name="c", subcore_axis_name="s")


@functools.partial(
    pl.kernel,
    mesh=_SC_MESH,
    out_type=(
        jax.ShapeDtypeStruct((N_IDS, D_ENTI), jnp.float32),
        jax.ShapeDtypeStruct((N_PAIR * TM_CHUNK,), jnp.int32),
    ),
    scratch_types=[
        pltpu.VMEM((G_ROWS,), jnp.int32),
        pltpu.VMEM((G_ROWS, D_ENTI), jnp.float32),
        pltpu.VMEM((TM_CHUNK,), jnp.int32),
        pltpu.SemaphoreType.DMA,
    ],
)
def _sc_gather(ids_hbm, enti_hbm, tm_hbm, rows_out, tm_out,
               idx_v, rows_v, tm_v, sem):
    wid = lax.axis_index("s") * 2 + lax.axis_index("c")

    @pl.when(wid < G_WORKERS)
    def _():
        base = wid * G_ROWS
        pltpu.sync_copy(ids_hbm.at[pl.ds(base, G_ROWS)], idx_v)
        pltpu.async_copy(enti_hbm.at[idx_v], rows_v, sem).wait()
        pltpu.sync_copy(rows_v, rows_out.at[pl.ds(base, G_ROWS)])

    pltpu.sync_copy(tm_hbm, tm_v)
    pltpu.sync_copy(tm_v, tm_out.at[pl.ds(wid * TM_CHUNK, TM_CHUNK)])


def _assemble_body(g_ref, sW1_ref, sb1_ref, sW2_ref, oW1_ref, ob1_ref,
                   oW2_ref, sctx_in_ref, octx_in_ref,
                   pre_ref, suf_ref, subj_ref, obj_ref,
                   sctx_scr, octx_scr):
    cc = pl.program_id(0)
    p = pl.program_id(1)

    @pl.when((cc == 0) & (p == 0))
    def _():
        s_e = g_ref[0:N_PAIR]
        o_e = g_ref[N_PAIR:N_IDS]
        s_h = jnp.maximum(
            jnp.dot(s_e, sW1_ref[:], preferred_element_type=jnp.float32)
            + sb1_ref[:], 0.0)
        o_h = jnp.maximum(
            jnp.dot(o_e, oW1_ref[:], preferred_element_type=jnp.float32)
            + ob1_ref[:], 0.0)
        s_emb = jnp.dot(s_h, sW2_ref[:], preferred_element_type=jnp.float32)
        o_emb = jnp.dot(o_h, oW2_ref[:], preferred_element_type=jnp.float32)
        sctx_scr[:] = sctx_in_ref[:][None, :, :] + s_emb[:, None, :]
        octx_scr[:] = octx_in_ref[:][None, :, :] + o_emb[:, None, :]

    s_ctx = jnp.broadcast_to(sctx_scr[p][None, :, :], (C_BLK, N_CTX, D))
    o_ctx = jnp.broadcast_to(octx_scr[p][None, :, :], (C_BLK, N_CTX, D))
    pre = pre_ref[:]
    suf = suf_ref[:]
    subj_ref[:, 0:1, :] = pre
    subj_ref[:, 1:1 + N_CTX, :] = s_ctx
    subj_ref[:, 1 + N_CTX:MAX_L, :] = suf
    obj_ref[:, 0:1, :] = pre
    obj_ref[:, 1:1 + N_CTX, :] = o_ctx
    obj_ref[:, 1 + N_CTX:MAX_L, :] = suf


def kernel(so_cls_ids, enti_txt_embds, prefix_embds, suffix_embds, token_mask,
           subj_ctx_embds, obj_ctx_embds, sW1, sb1, sW2, oW1, ob1, oW2):
    prefix_sl = prefix_embds[1:N_CLS + 1]
    suffix_sl = suffix_embds[1:N_CLS + 1]
    tm_flat = token_mask[1:N_CLS + 1].reshape(-1)
    ids_flat = so_cls_ids.T.reshape(-1)

    gathered, tm_rep_flat = _sc_gather(ids_flat, enti_txt_embds, tm_flat)

    const2 = lambda cc, p: (0, 0)
    subj, obj = pl.pallas_call(
        _assemble_body,
        grid=(NCC, N_PAIR),
        in_specs=[
            pl.BlockSpec((N_IDS, D_ENTI), const2),
            pl.BlockSpec((D_ENTI, D_ENTI), const2),
            pl.BlockSpec((D_ENTI,), lambda cc, p: (0,)),
            pl.BlockSpec((D_ENTI, D), const2),
            pl.BlockSpec((D_ENTI, D_ENTI), const2),
            pl.BlockSpec((D_ENTI,), lambda cc, p: (0,)),
            pl.BlockSpec((D_ENTI, D), const2),
            pl.BlockSpec((N_CTX, D), const2),
            pl.BlockSpec((N_CTX, D), const2),
            pl.BlockSpec((C_BLK, 1, D), lambda cc, p: (cc, 0, 0)),
            pl.BlockSpec((C_BLK, SUF_L, D), lambda cc, p: (cc, 0, 0)),
        ],
        out_specs=[
            pl.BlockSpec((C_BLK, MAX_L, D), lambda cc, p: (p * NCC + cc, 0, 0)),
            pl.BlockSpec((C_BLK, MAX_L, D), lambda cc, p: (p * NCC + cc, 0, 0)),
        ],
        out_shape=(
            jax.ShapeDtypeStruct((N_PAIR * N_CLS, MAX_L, D), jnp.float32),
            jax.ShapeDtypeStruct((N_PAIR * N_CLS, MAX_L, D), jnp.float32),
        ),
        scratch_shapes=[
            pltpu.VMEM((N_PAIR, N_CTX, D), jnp.float32),
            pltpu.VMEM((N_PAIR, N_CTX, D), jnp.float32),
        ],
    )(gathered, sW1, sb1, sW2, oW1, ob1, oW2, subj_ctx_embds, obj_ctx_embds,
      prefix_sl, suffix_sl)

    return subj, obj, tm_rep_flat.reshape(N_PAIR * N_CLS, MAX_L)


# speedup vs baseline: 1.0057x; 1.0005x over previous
import functools

import jax
import jax.numpy as jnp
from jax import lax
from jax.experimental import pallas as pl
from jax.experimental.pallas import tpu as pltpu
from jax.experimental.pallas import tpu_sc as plsc

N_PAIR = 32
N_CTX = 8
MAX_L = 40
D = 768
NUM_BASE = 92
NUM_NOVEL = 40
N_CLS = NUM_BASE + NUM_NOVEL
VOCAB = 1000
D_ENTI = 256
SUF_L = MAX_L - 1 - N_CTX

C_BLK = 66
NCC = N_CLS // C_BLK

N_IDS = 2 * N_PAIR
G_WORKERS = 8
G_ROWS = N_IDS // G_WORKERS
TM_CHUNK = N_CLS * MAX_L

_SC_MESH = plsc.VectorSubcoreMesh(core_axis_name="c", subcore_axis_name="s")


@functools.partial(
    pl.kernel,
    mesh=_SC_MESH,
    out_type=jax.ShapeDtypeStruct((N_IDS, D_ENTI), jnp.float32),
    scratch_types=[
        pltpu.VMEM((G_ROWS,), jnp.int32),
        pltpu.VMEM((G_ROWS, D_ENTI), jnp.float32),
        pltpu.SemaphoreType.DMA,
    ],
)
def _sc_gather(ids_hbm, enti_hbm, rows_out, idx_v, rows_v, sem):
    wid = lax.axis_index("s") * 2 + lax.axis_index("c")

    @pl.when(wid < G_WORKERS)
    def _():
        base = wid * G_ROWS
        pltpu.sync_copy(ids_hbm.at[pl.ds(base, G_ROWS)], idx_v)
        pltpu.async_copy(enti_hbm.at[idx_v], rows_v, sem).wait()
        pltpu.sync_copy(rows_v, rows_out.at[pl.ds(base, G_ROWS)])


@functools.partial(
    pl.kernel,
    mesh=_SC_MESH,
    out_type=jax.ShapeDtypeStruct((N_PAIR * TM_CHUNK,), jnp.int32),
    scratch_types=[pltpu.VMEM((TM_CHUNK,), jnp.int32)],
)
def _sc_mask(tm_hbm, rows_hbm, tm_out, tm_v):
    del rows_hbm
    wid = lax.axis_index("s") * 2 + lax.axis_index("c")
    pltpu.sync_copy(tm_hbm, tm_v)
    pltpu.sync_copy(tm_v, tm_out.at[pl.ds(wid * TM_CHUNK, TM_CHUNK)])


def _assemble_body(g_ref, sW1_ref, sb1_ref, sW2_ref, oW1_ref, ob1_ref,
                   oW2_ref, sctx_in_ref, octx_in_ref,
                   pre_ref, suf_ref, subj_ref, obj_ref,
                   sctx_scr, octx_scr):
    cc = pl.program_id(0)
    p = pl.program_id(1)

    @pl.when((cc == 0) & (p == 0))
    def _():
        s_e = g_ref[0:N_PAIR]
        o_e = g_ref[N_PAIR:N_IDS]
        s_h = jnp.maximum(
            jnp.dot(s_e, sW1_ref[:], preferred_element_type=jnp.float32)
            + sb1_ref[:], 0.0)
        o_h = jnp.maximum(
            jnp.dot(o_e, oW1_ref[:], preferred_element_type=jnp.float32)
            + ob1_ref[:], 0.0)
        s_emb = jnp.dot(s_h, sW2_ref[:], preferred_element_type=jnp.float32)
        o_emb = jnp.dot(o_h, oW2_ref[:], preferred_element_type=jnp.float32)
        sctx_scr[:] = sctx_in_ref[:][None, :, :] + s_emb[:, None, :]
        octx_scr[:] = octx_in_ref[:][None, :, :] + o_emb[:, None, :]

    s_ctx = jnp.broadcast_to(sctx_scr[p][None, :, :], (C_BLK, N_CTX, D))
    o_ctx = jnp.broadcast_to(octx_scr[p][None, :, :], (C_BLK, N_CTX, D))
    pre = pre_ref[:]
    suf = suf_ref[:]
    subj_ref[:, 0:1, :] = pre
    subj_ref[:, 1:1 + N_CTX, :] = s_ctx
    subj_ref[:, 1 + N_CTX:MAX_L, :] = suf
    obj_ref[:, 0:1, :] = pre
    obj_ref[:, 1:1 + N_CTX, :] = o_ctx
    obj_ref[:, 1 + N_CTX:MAX_L, :] = suf


def kernel(so_cls_ids, enti_txt_embds, prefix_embds, suffix_embds, token_mask,
           subj_ctx_embds, obj_ctx_embds, sW1, sb1, sW2, oW1, ob1, oW2):
    prefix_sl = prefix_embds[1:N_CLS + 1]
    suffix_sl = suffix_embds[1:N_CLS + 1]
    tm_flat = token_mask[1:N_CLS + 1].reshape(-1)
    ids_flat = so_cls_ids.T.reshape(-1)

    gathered = _sc_gather(ids_flat, enti_txt_embds)
    tm_rep_flat = _sc_mask(tm_flat, gathered)

    const2 = lambda cc, p: (0, 0)
    subj, obj = pl.pallas_call(
        _assemble_body,
        grid=(NCC, N_PAIR),
        in_specs=[
            pl.BlockSpec((N_IDS, D_ENTI), const2),
            pl.BlockSpec((D_ENTI, D_ENTI), const2),
            pl.BlockSpec((D_ENTI,), lambda cc, p: (0,)),
            pl.BlockSpec((D_ENTI, D), const2),
            pl.BlockSpec((D_ENTI, D_ENTI), const2),
            pl.BlockSpec((D_ENTI,), lambda cc, p: (0,)),
            pl.BlockSpec((D_ENTI, D), const2),
            pl.BlockSpec((N_CTX, D), const2),
            pl.BlockSpec((N_CTX, D), const2),
            pl.BlockSpec((C_BLK, 1, D), lambda cc, p: (cc, 0, 0)),
            pl.BlockSpec((C_BLK, SUF_L, D), lambda cc, p: (cc, 0, 0)),
        ],
        out_specs=[
            pl.BlockSpec((C_BLK, MAX_L, D), lambda cc, p: (p * NCC + cc, 0, 0)),
            pl.BlockSpec((C_BLK, MAX_L, D), lambda cc, p: (p * NCC + cc, 0, 0)),
        ],
        out_shape=(
            jax.ShapeDtypeStruct((N_PAIR * N_CLS, MAX_L, D), jnp.float32),
            jax.ShapeDtypeStruct((N_PAIR * N_CLS, MAX_L, D), jnp.float32),
        ),
        scratch_shapes=[
            pltpu.VMEM((N_PAIR, N_CTX, D), jnp.float32),
            pltpu.VMEM((N_PAIR, N_CTX, D), jnp.float32),
        ],
    )(gathered, sW1, sb1, sW2, oW1, ob1, oW2, subj_ctx_embds, obj_ctx_embds,
      prefix_sl, suffix_sl)

    return subj, obj, tm_rep_flat.reshape(N_PAIR * N_CLS, MAX_L)
